# Initial kernel scaffold; baseline (speedup 1.0000x reference)
import functools

import jax
import jax.numpy as jnp
from jax import lax
from jax.experimental import pallas as pl
from jax.experimental.pallas import tpu as pltpu
from jax.experimental.pallas import tpu_sc as plsc

N = 10000
NP = 10240
E = 320000
EP = 323584
NW = 32
CHUNK = 128
CPW = EP // (NW * CHUNK)
NTILES = 16
STRIPE = NP // NTILES
NG = 64
ROWBLK = 1280
GRID = NP // ROWBLK


def _sc_mesh():
    return plsc.VectorSubcoreMesh(core_axis_name="c", subcore_axis_name="s")


@functools.partial(
    pl.kernel,
    mesh=_sc_mesh(),
    out_type=jax.ShapeDtypeStruct((2, NP, 16), jnp.float32),
    scratch_types=[
        pltpu.VMEM((CPW, CHUNK), jnp.int32),
        pltpu.VMEM((CHUNK, 16), jnp.float32),
        pltpu.VMEM_SHARED((NP, 16), jnp.float32),
    ],
)
def _deg_kernel(dst_hbm, ones_hbm, z16_hbm, out_hbm, dst_v, ones_v, acc_sh):
    cid = lax.axis_index("c")
    sid = lax.axis_index("s")
    wid = sid * 2 + cid
    pltpu.sync_copy(z16_hbm, acc_sh.at[pl.ds(sid * STRIPE, STRIPE)])
    pltpu.sync_copy(dst_hbm.at[wid], dst_v)
    pltpu.sync_copy(ones_hbm, ones_v)
    plsc.subcore_barrier()

    def body(j, carry):
        pltpu.sync_copy(ones_v, acc_sh.at[dst_v.at[j]], add=True)
        return carry

    lax.fori_loop(0, CPW, body, 0)
    plsc.subcore_barrier()
    pltpu.sync_copy(
        acc_sh.at[pl.ds(sid * STRIPE, STRIPE)],
        out_hbm.at[cid, pl.ds(sid * STRIPE, STRIPE)],
    )


@functools.partial(
    pl.kernel,
    mesh=_sc_mesh(),
    out_type=jax.ShapeDtypeStruct((2, NP, 128), jnp.float32),
    scratch_types=[
        pltpu.VMEM((CPW, CHUNK), jnp.int32),
        pltpu.VMEM((CPW, CHUNK), jnp.int32),
        pltpu.VMEM((CHUNK, 128), jnp.float32),
        pltpu.VMEM_SHARED((NP, 128), jnp.float32),
        pltpu.SemaphoreType.DMA,
    ],
)
def _edge_kernel(src_hbm, dst_hbm, g_hbm, z_hbm, out_hbm,
                 src_v, dst_v, rows_v, acc_sh, sem):
    cid = lax.axis_index("c")
    sid = lax.axis_index("s")
    wid = sid * 2 + cid
    pltpu.sync_copy(z_hbm, acc_sh.at[pl.ds(sid * STRIPE, STRIPE)])
    pltpu.sync_copy(src_hbm.at[wid], src_v)
    pltpu.sync_copy(dst_hbm.at[wid], dst_v)
    plsc.subcore_barrier()

    def body(j, carry):
        pltpu.async_copy(g_hbm.at[src_v.at[j]], rows_v, sem).wait()
        pltpu.sync_copy(rows_v, acc_sh.at[dst_v.at[j]], add=True)
        return carry

    lax.fori_loop(0, CPW, body, 0)
    plsc.subcore_barrier()
    pltpu.sync_copy(
        acc_sh.at[pl.ds(sid * STRIPE, STRIPE)],
        out_hbm.at[cid, pl.ds(sid * STRIPE, STRIPE)],
    )


def _tc_first(counts, x_pad, W1):
    def body(c_ref, x_ref, w_ref, dinv_ref, g_ref):
        c = c_ref[0, :, 0:1] + c_ref[1, :, 0:1]
        dinv = lax.rsqrt(c + 1.0)
        dinv_b = jnp.broadcast_to(dinv, (ROWBLK, 128))
        h = jnp.dot(x_ref[...], w_ref[...], preferred_element_type=jnp.float32)
        dinv_ref[...] = dinv_b
        g_ref[...] = dinv_b * h

    return pl.pallas_call(
        body,
        grid=(GRID,),
        in_specs=[
            pl.BlockSpec((2, ROWBLK, 16), lambda i: (0, i, 0)),
            pl.BlockSpec((ROWBLK, 128), lambda i: (i, 0)),
            pl.BlockSpec((128, 128), lambda i: (0, 0)),
        ],
        out_specs=[
            pl.BlockSpec((ROWBLK, 128), lambda i: (i, 0)),
            pl.BlockSpec((ROWBLK, 128), lambda i: (i, 0)),
        ],
        out_shape=[jax.ShapeDtypeStruct((NP, 128), jnp.float32)] * 2,
    )(counts, x_pad, W1)


def _tc_layer(acc, g, dinv_b, b_row, W_next):
    def body(a_ref, g_ref, d_ref, b_ref, w_ref, o_ref):
        y = d_ref[...] * (a_ref[0] + a_ref[1] + g_ref[...]) + b_ref[...]
        y = jnp.maximum(y, 0.0)
        h = jnp.dot(y, w_ref[...], preferred_element_type=jnp.float32)
        o_ref[...] = d_ref[...] * h

    return pl.pallas_call(
        body,
        grid=(GRID,),
        in_specs=[
            pl.BlockSpec((2, ROWBLK, 128), lambda i: (0, i, 0)),
            pl.BlockSpec((ROWBLK, 128), lambda i: (i, 0)),
            pl.BlockSpec((ROWBLK, 128), lambda i: (i, 0)),
            pl.BlockSpec((1, 128), lambda i: (0, 0)),
            pl.BlockSpec((128, 128), lambda i: (0, 0)),
        ],
        out_specs=pl.BlockSpec((ROWBLK, 128), lambda i: (i, 0)),
        out_shape=jax.ShapeDtypeStruct((NP, 128), jnp.float32),
    )(acc, g, dinv_b, b_row, W_next)


def _tc_final(acc, g, dinv_b, b_row, batch_b, Wl, bl_row):
    def body(a_ref, g_ref, d_ref, b_ref, bt_ref, wl_ref, bl_ref, o_ref,
             sums, cnts):
        i = pl.program_id(0)

        @pl.when(i == 0)
        def _init():
            sums[...] = jnp.zeros_like(sums)
            cnts[...] = jnp.zeros_like(cnts)

        y = d_ref[...] * (a_ref[0] + a_ref[1] + g_ref[...]) + b_ref[...]
        lanes = lax.broadcasted_iota(jnp.int32, (ROWBLK, 128), 1)
        mt = (bt_ref[...] == lanes).astype(jnp.float32)
        dn = (((0,), (0,)), ((), ()))
        sums[...] += lax.dot_general(mt, y, dn,
                                     preferred_element_type=jnp.float32)
        cnts[...] += lax.dot_general(mt, jnp.ones_like(y), dn,
                                     preferred_element_type=jnp.float32)

        @pl.when(i == GRID - 1)
        def _finish():
            pooled = sums[...] / jnp.maximum(cnts[...], 1.0)
            p = pooled[0:NG, :]
            o_ref[...] = (
                jnp.dot(p, wl_ref[...], preferred_element_type=jnp.float32)
                + bl_ref[...]
            )

    return pl.pallas_call(
        body,
        grid=(GRID,),
        in_specs=[
            pl.BlockSpec((2, ROWBLK, 128), lambda i: (0, i, 0)),
            pl.BlockSpec((ROWBLK, 128), lambda i: (i, 0)),
            pl.BlockSpec((ROWBLK, 128), lambda i: (i, 0)),
            pl.BlockSpec((1, 128), lambda i: (0, 0)),
            pl.BlockSpec((ROWBLK, 128), lambda i: (i, 0)),
            pl.BlockSpec((128, 16), lambda i: (0, 0)),
            pl.BlockSpec((1, 16), lambda i: (0, 0)),
        ],
        out_specs=pl.BlockSpec((NG, 16), lambda i: (0, 0)),
        out_shape=jax.ShapeDtypeStruct((NG, 16), jnp.float32),
        scratch_shapes=[
            pltpu.VMEM((128, 128), jnp.float32),
            pltpu.VMEM((128, 128), jnp.float32),
        ],
    )(acc, g, dinv_b, b_row, batch_b, Wl, bl_row)


def kernel(x, edge_index, batch, W1, b1, W2, b2, W3, b3, Wl, bl):
    src = edge_index[0].astype(jnp.int32)
    dst = edge_index[1].astype(jnp.int32)
    pad = jnp.full((EP - E,), N, jnp.int32)
    src_p = jnp.concatenate([src, pad]).reshape(NW, CPW, CHUNK)
    dst_p = jnp.concatenate([dst, pad]).reshape(NW, CPW, CHUNK)
    x_pad = jnp.pad(x, ((0, NP - N), (0, 0)))
    batch_b = jnp.broadcast_to(
        jnp.pad(batch.astype(jnp.int32), (0, NP - N), constant_values=NG)[:, None],
        (NP, 128),
    )
    ones16 = jnp.ones((CHUNK, 16), jnp.float32)
    z16 = jnp.zeros((STRIPE, 16), jnp.float32)
    z128 = jnp.zeros((STRIPE, 128), jnp.float32)

    counts = _deg_kernel(dst_p, ones16, z16)
    dinv_b, g1 = _tc_first(counts, x_pad, W1)
    acc1 = _edge_kernel(src_p, dst_p, g1, z128)
    g2 = _tc_layer(acc1, g1, dinv_b, b1.reshape(1, 128), W2)
    acc2 = _edge_kernel(src_p, dst_p, g2, z128)
    g3 = _tc_layer(acc2, g2, dinv_b, b2.reshape(1, 128), W3)
    acc3 = _edge_kernel(src_p, dst_p, g3, z128)
    return _tc_final(acc3, g3, dinv_b, b3.reshape(1, 128), batch_b,
                     Wl, bl.reshape(1, 16))


# speedup vs baseline: 9.8099x; 9.8099x over previous
import functools

import jax
import jax.numpy as jnp
from jax import lax
from jax.experimental import pallas as pl
from jax.experimental.pallas import tpu as pltpu
from jax.experimental.pallas import tpu_sc as plsc

N = 10000
NP = 10240
E = 320000
EP = 323584
NW = 32
CHUNK = 128
CPW = EP // (NW * CHUNK)
NTILES = 16
STRIPE = NP // NTILES
NG = 64
ROWBLK = 1280
GRID = NP // ROWBLK


def _sc_mesh():
    return plsc.VectorSubcoreMesh(core_axis_name="c", subcore_axis_name="s")


@functools.partial(
    pl.kernel,
    mesh=_sc_mesh(),
    out_type=jax.ShapeDtypeStruct((2, NP, 128), jnp.float32),
    scratch_types=[
        pltpu.VMEM((CPW, CHUNK), jnp.int32),
        pltpu.VMEM((CHUNK, 128), jnp.float32),
        pltpu.VMEM_SHARED((NP, 128), jnp.float32),
    ],
)
def _deg_kernel(dst_hbm, ones_hbm, z_hbm, out_hbm, dst_v, ones_v, acc_sh):
    cid = lax.axis_index("c")
    sid = lax.axis_index("s")
    wid = sid * 2 + cid
    pltpu.sync_copy(z_hbm, acc_sh.at[pl.ds(sid * STRIPE, STRIPE)])
    pltpu.sync_copy(dst_hbm.at[wid], dst_v)
    pltpu.sync_copy(ones_hbm, ones_v)
    plsc.subcore_barrier()

    def body(j, carry):
        pltpu.sync_copy(ones_v, acc_sh.at[dst_v.at[j]], add=True)
        return carry

    lax.fori_loop(0, CPW, body, 0)
    plsc.subcore_barrier()
    pltpu.sync_copy(
        acc_sh.at[pl.ds(sid * STRIPE, STRIPE)],
        out_hbm.at[cid, pl.ds(sid * STRIPE, STRIPE)],
    )


@functools.partial(
    pl.kernel,
    mesh=_sc_mesh(),
    out_type=jax.ShapeDtypeStruct((2, NP, 128), jnp.float32),
    scratch_types=[
        pltpu.VMEM((CPW, CHUNK), jnp.int32),
        pltpu.VMEM((CPW, CHUNK), jnp.int32),
        pltpu.VMEM((CHUNK, 128), jnp.float32),
        pltpu.VMEM_SHARED((NP, 128), jnp.float32),
        pltpu.SemaphoreType.DMA,
    ],
)
def _edge_kernel(src_hbm, dst_hbm, g_hbm, z_hbm, out_hbm,
                 src_v, dst_v, rows_v, acc_sh, sem):
    cid = lax.axis_index("c")
    sid = lax.axis_index("s")
    wid = sid * 2 + cid
    pltpu.sync_copy(z_hbm, acc_sh.at[pl.ds(sid * STRIPE, STRIPE)])
    pltpu.sync_copy(src_hbm.at[wid], src_v)
    pltpu.sync_copy(dst_hbm.at[wid], dst_v)
    plsc.subcore_barrier()

    def body(j, carry):
        pltpu.async_copy(g_hbm.at[src_v.at[j]], rows_v, sem).wait()
        pltpu.sync_copy(rows_v, acc_sh.at[dst_v.at[j]], add=True)
        return carry

    lax.fori_loop(0, CPW, body, 0)
    plsc.subcore_barrier()
    pltpu.sync_copy(
        acc_sh.at[pl.ds(sid * STRIPE, STRIPE)],
        out_hbm.at[cid, pl.ds(sid * STRIPE, STRIPE)],
    )


def _tc_first(counts, x_pad, W1):
    def body(c_ref, x_ref, w_ref, dinv_ref, g_ref):
        c = c_ref[0, :, 0:1] + c_ref[1, :, 0:1]
        dinv = lax.rsqrt(c + 1.0)
        dinv_b = jnp.broadcast_to(dinv, (ROWBLK, 128))
        h = jnp.dot(x_ref[...], w_ref[...], preferred_element_type=jnp.float32)
        dinv_ref[...] = dinv_b
        g_ref[...] = dinv_b * h

    return pl.pallas_call(
        body,
        grid=(GRID,),
        in_specs=[
            pl.BlockSpec((2, ROWBLK, 128), lambda i: (0, i, 0)),
            pl.BlockSpec((ROWBLK, 128), lambda i: (i, 0)),
            pl.BlockSpec((128, 128), lambda i: (0, 0)),
        ],
        out_specs=[
            pl.BlockSpec((ROWBLK, 128), lambda i: (i, 0)),
            pl.BlockSpec((ROWBLK, 128), lambda i: (i, 0)),
        ],
        out_shape=[jax.ShapeDtypeStruct((NP, 128), jnp.float32)] * 2,
    )(counts, x_pad, W1)


def _tc_layer(acc, g, dinv_b, b_row, W_next):
    def body(a_ref, g_ref, d_ref, b_ref, w_ref, o_ref):
        y = d_ref[...] * (a_ref[0] + a_ref[1] + g_ref[...]) + b_ref[...]
        y = jnp.maximum(y, 0.0)
        h = jnp.dot(y, w_ref[...], preferred_element_type=jnp.float32)
        o_ref[...] = d_ref[...] * h

    return pl.pallas_call(
        body,
        grid=(GRID,),
        in_specs=[
            pl.BlockSpec((2, ROWBLK, 128), lambda i: (0, i, 0)),
            pl.BlockSpec((ROWBLK, 128), lambda i: (i, 0)),
            pl.BlockSpec((ROWBLK, 128), lambda i: (i, 0)),
            pl.BlockSpec((1, 128), lambda i: (0, 0)),
            pl.BlockSpec((128, 128), lambda i: (0, 0)),
        ],
        out_specs=pl.BlockSpec((ROWBLK, 128), lambda i: (i, 0)),
        out_shape=jax.ShapeDtypeStruct((NP, 128), jnp.float32),
    )(acc, g, dinv_b, b_row, W_next)


def _tc_final(acc, g, dinv_b, b_row, batch_b, Wl, bl_row):
    def body(a_ref, g_ref, d_ref, b_ref, bt_ref, wl_ref, bl_ref, o_ref,
             sums, cnts):
        i = pl.program_id(0)

        @pl.when(i == 0)
        def _init():
            sums[...] = jnp.zeros_like(sums)
            cnts[...] = jnp.zeros_like(cnts)

        y = d_ref[...] * (a_ref[0] + a_ref[1] + g_ref[...]) + b_ref[...]
        lanes = lax.broadcasted_iota(jnp.int32, (ROWBLK, 128), 1)
        mt = (bt_ref[...] == lanes).astype(jnp.float32)
        dn = (((0,), (0,)), ((), ()))
        sums[...] += lax.dot_general(mt, y, dn,
                                     preferred_element_type=jnp.float32)
        cnts[...] += lax.dot_general(mt, jnp.ones_like(y), dn,
                                     preferred_element_type=jnp.float32)

        @pl.when(i == GRID - 1)
        def _finish():
            pooled = sums[...] / jnp.maximum(cnts[...], 1.0)
            p = pooled[0:NG, :]
            o_ref[...] = (
                jnp.dot(p, wl_ref[...], preferred_element_type=jnp.float32)
                + bl_ref[...]
            )

    return pl.pallas_call(
        body,
        grid=(GRID,),
        in_specs=[
            pl.BlockSpec((2, ROWBLK, 128), lambda i: (0, i, 0)),
            pl.BlockSpec((ROWBLK, 128), lambda i: (i, 0)),
            pl.BlockSpec((ROWBLK, 128), lambda i: (i, 0)),
            pl.BlockSpec((1, 128), lambda i: (0, 0)),
            pl.BlockSpec((ROWBLK, 128), lambda i: (i, 0)),
            pl.BlockSpec((128, 16), lambda i: (0, 0)),
            pl.BlockSpec((1, 16), lambda i: (0, 0)),
        ],
        out_specs=pl.BlockSpec((NG, 16), lambda i: (0, 0)),
        out_shape=jax.ShapeDtypeStruct((NG, 16), jnp.float32),
        scratch_shapes=[
            pltpu.VMEM((128, 128), jnp.float32),
            pltpu.VMEM((128, 128), jnp.float32),
        ],
    )(acc, g, dinv_b, b_row, batch_b, Wl, bl_row)


def kernel(x, edge_index, batch, W1, b1, W2, b2, W3, b3, Wl, bl):
    src = edge_index[0].astype(jnp.int32)
    dst = edge_index[1].astype(jnp.int32)
    pad = jnp.full((EP - E,), N, jnp.int32)
    src_p = jnp.concatenate([src, pad]).reshape(NW, CPW, CHUNK)
    dst_p = jnp.concatenate([dst, pad]).reshape(NW, CPW, CHUNK)
    x_pad = jnp.pad(x, ((0, NP - N), (0, 0)))
    batch_b = jnp.broadcast_to(
        jnp.pad(batch.astype(jnp.int32), (0, NP - N), constant_values=NG)[:, None],
        (NP, 128),
    )
    ones128 = jnp.ones((CHUNK, 128), jnp.float32)
    z128 = jnp.zeros((STRIPE, 128), jnp.float32)

    counts = _deg_kernel(dst_p, ones128, z128)
    dinv_b, g1 = _tc_first(counts, x_pad, W1)
    acc1 = _edge_kernel(src_p, dst_p, g1, z128)
    g2 = _tc_layer(acc1, g1, dinv_b, b1.reshape(1, 128), W2)
    acc2 = _edge_kernel(src_p, dst_p, g2, z128)
    g3 = _tc_layer(acc2, g2, dinv_b, b2.reshape(1, 128), W3)
    acc3 = _edge_kernel(src_p, dst_p, g3, z128)
    return _tc_final(acc3, g3, dinv_b, b3.reshape(1, 128), batch_b,
                     Wl, bl.reshape(1, 16))
